# baseline (speedup 1.0000x reference)
import functools

import jax
import jax.numpy as jnp
from jax import lax
from jax.experimental import pallas as pl
from jax.experimental.pallas import tpu as pltpu
from jax.experimental.pallas import tpu_sc as plsc

B = 16384
D = 32
NC, NS = 2, 16
NW = NC * NS
BPW = B // NW

_sc_mesh = plsc.VectorSubcoreMesh(core_axis_name="c", subcore_axis_name="s")


def _build_gather4():
    out_t = [jax.ShapeDtypeStruct((B, D), jnp.float32)] * 4
    scratch = (
        [pltpu.VMEM((BPW,), jnp.int32)]
        + [pltpu.SemaphoreType.DMA for _ in range(4)]
    )

    @functools.partial(pl.kernel, mesh=_sc_mesh, out_type=out_t,
                       scratch_types=scratch)
    def gather4(ct, st, gt, mt, ci, si, gi, mi,
                o0, o1, o2, o3,
                iv, s0, s1, s2, s3):
        wid = lax.axis_index("s") * NC + lax.axis_index("c")
        base = wid * BPW
        sl = pl.ds(base, BPW)
        tables = (ct, st, gt, mt)
        idx_hbm = (ci, si, gi, mi)
        sems = (s0, s1, s2, s3)
        outs = (o0, o1, o2, o3)
        for tbl, ih, sem, o in zip(tables, idx_hbm, sems, outs):
            pltpu.sync_copy(ih.at[sl], iv)

            @pl.loop(0, BPW, step=16)
            def _(i, tbl=tbl, sem=sem, o=o):
                v = iv[pl.ds(i, 16)]
                for j in range(16):
                    pltpu.async_copy(tbl.at[v[j]], o.at[base + i + j], sem)
        for tbl, sem, o in zip(tables, sems, outs):
            pltpu.make_async_copy(tbl.at[pl.ds(0, BPW)], o.at[sl], sem).wait()

    return gather4


_gather4 = _build_gather4()

BM = 2048


def _mlp_body(e0, e1, e2, e3, c2d, t2d, cw, cb, tw, tb,
              w1, b1, w2, b2, w3, b3, out):
    cost_e = c2d[...] * cw[...] + cb[...]
    time_e = t2d[...] * tw[...] + tb[...]
    x = jnp.concatenate(
        [e0[...], e1[...], e2[...], e3[...], cost_e, time_e], axis=1)
    h = jnp.maximum(
        jnp.dot(x, w1[...], preferred_element_type=jnp.float32) + b1[...], 0.0)
    h = jnp.maximum(
        jnp.dot(h, w2[...], preferred_element_type=jnp.float32) + b2[...], 0.0)
    out[...] = jnp.dot(h, w3[...], preferred_element_type=jnp.float32) + b3[...]


def _full(shape):
    return pl.BlockSpec(shape, lambda i: (0, 0))


_mlp = pl.pallas_call(
    _mlp_body,
    grid=(B // BM,),
    in_specs=[
        pl.BlockSpec((BM, D), lambda i: (i, 0)),
        pl.BlockSpec((BM, D), lambda i: (i, 0)),
        pl.BlockSpec((BM, D), lambda i: (i, 0)),
        pl.BlockSpec((BM, D), lambda i: (i, 0)),
        pl.BlockSpec((BM, 1), lambda i: (i, 0)),
        pl.BlockSpec((BM, 1), lambda i: (i, 0)),
        _full((1, D)),
        _full((1, D)),
        _full((1, D)),
        _full((1, D)),
        _full((6 * D, 256)),
        _full((1, 256)),
        _full((256, 128)),
        _full((1, 128)),
        _full((128, 32)),
        _full((1, 32)),
    ],
    out_specs=pl.BlockSpec((BM, 32), lambda i: (i, 0)),
    out_shape=jax.ShapeDtypeStruct((B, 32), jnp.float32),
)


def kernel(cost, time, center_idx, subject_idx, grade_idx, method_idx,
           center_table, subject_table, grade_table, method_table,
           cost_W, cost_b, time_W, time_b, W1, b1, W2, b2, W3, b3):
    e0, e1, e2, e3 = _gather4(
        center_table, subject_table, grade_table, method_table,
        center_idx, subject_idx, grade_idx, method_idx)
    return _mlp(
        e0, e1, e2, e3,
        cost[:, None].astype(jnp.float32), time[:, None].astype(jnp.float32),
        cost_W, cost_b[None, :], time_W, time_b[None, :],
        W1, b1[None, :], W2, b2[None, :], W3, b3[None, :])


# speedup vs baseline: 3.4625x; 3.4625x over previous
import functools

import jax
import jax.numpy as jnp
from jax import lax
from jax.experimental import pallas as pl
from jax.experimental.pallas import tpu as pltpu
from jax.experimental.pallas import tpu_sc as plsc

B = 16384
D = 32
NC, NS = 2, 16
NW = NC * NS
BPW = B // NW

_sc_mesh = plsc.VectorSubcoreMesh(core_axis_name="c", subcore_axis_name="s")


def _build_gather4():
    out_t = [jax.ShapeDtypeStruct((B, D), jnp.float32)] * 4
    chunk = BPW // 2
    scratch = (
        [pltpu.VMEM((BPW,), jnp.int32) for _ in range(4)]
        + [pltpu.VMEM((chunk, D), jnp.float32) for _ in range(2)]
        + [pltpu.SemaphoreType.DMA for _ in range(2)]
    )

    @functools.partial(pl.kernel, mesh=_sc_mesh, out_type=out_t,
                       scratch_types=scratch)
    def gather4(ct, st, gt, mt, ci, si, gi, mi,
                o0, o1, o2, o3,
                i0, i1, i2, i3, ra, rb, sa, sb):
        wid = lax.axis_index("s") * NC + lax.axis_index("c")
        base = wid * BPW
        sl = pl.ds(base, BPW)
        tables = (ct, st, gt, mt)
        idx_vmem = (i0, i1, i2, i3)
        outs = (o0, o1, o2, o3)
        bufs = (ra, rb)
        sems = (sa, sb)
        for ih, iv in zip((ci, si, gi, mi), idx_vmem):
            pltpu.sync_copy(ih.at[sl], iv)

        def fire(n):
            iv = idx_vmem[n // 2]
            tbl = tables[n // 2]
            c = (n % 2) * chunk
            b = n % 2

            @pl.loop(0, chunk, step=16)
            def _(i, tbl=tbl, iv=iv, c=c, b=b):
                v = iv[pl.ds(c + i, 16)]
                for j in range(16):
                    pltpu.async_copy(tbl.at[v[j]], bufs[b].at[i + j],
                                     sems[b])

        fire(0)
        fire(1)
        for n in range(8):
            b = n % 2
            pltpu.make_async_copy(tables[0].at[pl.ds(0, chunk)], bufs[b],
                                  sems[b]).wait()
            o = outs[n // 2]
            c = (n % 2) * chunk
            pltpu.sync_copy(bufs[b], o.at[pl.ds(base + c, chunk)])
            if n + 2 < 8:
                fire(n + 2)

    return gather4


_gather4 = _build_gather4()

BM = 2048


def _mlp_body(e0, e1, e2, e3, c2d, t2d, cw, cb, tw, tb,
              w1, b1, w2, b2, w3, b3, out):
    cost_e = c2d[...] * cw[...] + cb[...]
    time_e = t2d[...] * tw[...] + tb[...]
    x = jnp.concatenate(
        [e0[...], e1[...], e2[...], e3[...], cost_e, time_e], axis=1)
    h = jnp.maximum(
        jnp.dot(x, w1[...], preferred_element_type=jnp.float32) + b1[...], 0.0)
    h = jnp.maximum(
        jnp.dot(h, w2[...], preferred_element_type=jnp.float32) + b2[...], 0.0)
    out[...] = jnp.dot(h, w3[...], preferred_element_type=jnp.float32) + b3[...]


def _full(shape):
    return pl.BlockSpec(shape, lambda i: (0, 0))


_mlp = pl.pallas_call(
    _mlp_body,
    grid=(B // BM,),
    in_specs=[
        pl.BlockSpec((BM, D), lambda i: (i, 0)),
        pl.BlockSpec((BM, D), lambda i: (i, 0)),
        pl.BlockSpec((BM, D), lambda i: (i, 0)),
        pl.BlockSpec((BM, D), lambda i: (i, 0)),
        pl.BlockSpec((BM, 1), lambda i: (i, 0)),
        pl.BlockSpec((BM, 1), lambda i: (i, 0)),
        _full((1, D)),
        _full((1, D)),
        _full((1, D)),
        _full((1, D)),
        _full((6 * D, 256)),
        _full((1, 256)),
        _full((256, 128)),
        _full((1, 128)),
        _full((128, 32)),
        _full((1, 32)),
    ],
    out_specs=pl.BlockSpec((BM, 32), lambda i: (i, 0)),
    out_shape=jax.ShapeDtypeStruct((B, 32), jnp.float32),
)


def kernel(cost, time, center_idx, subject_idx, grade_idx, method_idx,
           center_table, subject_table, grade_table, method_table,
           cost_W, cost_b, time_W, time_b, W1, b1, W2, b2, W3, b3):
    e0, e1, e2, e3 = _gather4(
        center_table, subject_table, grade_table, method_table,
        center_idx, subject_idx, grade_idx, method_idx)
    return _mlp(
        e0, e1, e2, e3,
        cost[:, None].astype(jnp.float32), time[:, None].astype(jnp.float32),
        cost_W, cost_b[None, :], time_W, time_b[None, :],
        W1, b1[None, :], W2, b2[None, :], W3, b3[None, :])
